# baseline (speedup 1.0000x reference)
import functools

import jax
import jax.numpy as jnp
from jax import lax
from jax.experimental import pallas as pl
from jax.experimental.pallas import tpu as pltpu
from jax.experimental.pallas import tpu_sc as plsc

NSEG = 128
ROWS = 50000
D = 512
L = 16
NC = 2
NS = 16
NW = NC * NS
CH = 32
NCH = 48
PER_W = CH * NCH
SC_ROWS = NW * PER_W
TB = 2000
TBI = ROWS // TB - 1
TOFF = SC_ROWS - (ROWS - TB)


_MESH = plsc.VectorSubcoreMesh(core_axis_name="c", subcore_axis_name="s")


@functools.partial(
    pl.kernel,
    out_type=[
        jax.ShapeDtypeStruct((NW, NSEG, D), jnp.float32),
        jax.ShapeDtypeStruct((NW, NSEG, L), jnp.float32),
    ],
    mesh=_MESH,
    scratch_types=[
        pltpu.VMEM((PER_W,), jnp.int32),
        pltpu.VMEM((CH, D), jnp.float32),
        pltpu.VMEM((CH, D), jnp.float32),
        pltpu.VMEM((NSEG, D), jnp.float32),
        pltpu.VMEM((NSEG, L), jnp.float32),
        pltpu.SemaphoreType.DMA,
        pltpu.SemaphoreType.DMA,
        pltpu.SemaphoreType.DMA,
    ],
)
def _sc_segsum(x_hbm, b_hbm, sums_out, cnt_out,
               idx_v, xa, xb, acc, cnt, sem_a, sem_b, sem_i):
    cid = lax.axis_index("c")
    sid = lax.axis_index("s")
    wid = sid * NC + cid
    base = wid * PER_W

    z16 = jnp.zeros((L,), jnp.float32)
    o16 = jnp.ones((L,), jnp.float32)

    idx_cp = pltpu.async_copy(b_hbm.at[pl.ds(base, PER_W)], idx_v, sem_i)
    pltpu.async_copy(x_hbm.at[pl.ds(base, CH)], xa, sem_a)
    pltpu.async_copy(x_hbm.at[pl.ds(base + CH, CH)], xb, sem_b)

    def _zero(i, _):
        for cc in range(D // L):
            acc[i, pl.ds(cc * L, L)] = z16
        cnt[i, :] = z16
        return _

    lax.fori_loop(0, NSEG, _zero, None)
    idx_cp.wait()

    def _chunk(cur, idx_off):
        def grp(g, _):
            ids = idx_v[pl.ds(idx_off + g * L, L)]
            for u in range(L):
                s_ = ids[u]
                r = g * L + u
                for cc in range(D // L):
                    piece = cur[r, pl.ds(cc * L, L)]
                    plsc.addupdate(acc.at[s_, pl.ds(cc * L, L)], piece)
                plsc.addupdate(cnt.at[s_, pl.ds(0, L)], o16)
            return _

        lax.fori_loop(0, CH // L, grp, None)

    def _pair(p, _):
        c0 = 2 * p
        pltpu.make_async_copy(x_hbm.at[pl.ds(0, CH)], xa, sem_a).wait()
        _chunk(xa, c0 * CH)

        @pl.when(c0 + 2 < NCH)
        def _prefetch_a():
            pltpu.async_copy(
                x_hbm.at[pl.ds(base + (c0 + 2) * CH, CH)], xa, sem_a)

        pltpu.make_async_copy(x_hbm.at[pl.ds(0, CH)], xb, sem_b).wait()
        _chunk(xb, (c0 + 1) * CH)

        @pl.when(c0 + 3 < NCH)
        def _prefetch_b():
            pltpu.async_copy(
                x_hbm.at[pl.ds(base + (c0 + 3) * CH, CH)], xb, sem_b)

        return _

    lax.fori_loop(0, NCH // 2, _pair, None)

    pltpu.sync_copy(acc, sums_out.at[wid])
    pltpu.sync_copy(cnt, cnt_out.at[wid])


def _fin_body(sums_ref, cnt_ref, bt_ref, xt_ref, o_ref):
    s = jnp.sum(sums_ref[...], axis=0)
    c = jnp.sum(cnt_ref[...], axis=0)[:, 0:1]
    bt = bt_ref[0, 0, :]
    seg = lax.broadcasted_iota(jnp.int32, (NSEG, TB), 0)
    pos = lax.broadcasted_iota(jnp.int32, (NSEG, TB), 1)
    oh = ((seg == bt[None, :]) & (pos >= TOFF)).astype(jnp.float32)
    s = s + jnp.dot(oh, xt_ref[...], preferred_element_type=jnp.float32)
    c = c + jnp.sum(oh, axis=1, keepdims=True)
    o_ref[...] = s / jnp.maximum(c, 1.0)


def kernel(x, batch):
    b32 = batch.astype(jnp.int32)
    sums_p, cnt_p = _sc_segsum(x, b32)
    bt3 = b32.reshape(ROWS // TB, 1, TB)
    return pl.pallas_call(
        _fin_body,
        grid=(1,),
        in_specs=[
            pl.BlockSpec((NW, NSEG, D), lambda i: (0, 0, 0)),
            pl.BlockSpec((NW, NSEG, L), lambda i: (0, 0, 0)),
            pl.BlockSpec((1, 1, TB), lambda i: (TBI, 0, 0)),
            pl.BlockSpec((TB, D), lambda i: (TBI, 0)),
        ],
        out_specs=pl.BlockSpec((NSEG, D), lambda i: (0, 0)),
        out_shape=jax.ShapeDtypeStruct((NSEG, D), jnp.float32),
    )(sums_p, cnt_p, bt3, x)


# speedup vs baseline: 3.4849x; 3.4849x over previous
import functools

import jax
import jax.numpy as jnp
from jax import lax
from jax.experimental import pallas as pl
from jax.experimental.pallas import tpu as pltpu
from jax.experimental.pallas import tpu_sc as plsc

NSEG = 128
ROWS = 50000
D = 512
L = 16
NC = 2
NS = 16
NW = NC * NS
CH = 32
NCH = 8
PER_W = CH * NCH
SC_ROWS = NW * PER_W
R = 2000
B0 = SC_ROWS // R


_MESH = plsc.VectorSubcoreMesh(core_axis_name="c", subcore_axis_name="s")


@functools.partial(
    pl.kernel,
    out_type=[
        jax.ShapeDtypeStruct((NW, NSEG, D), jnp.float32),
        jax.ShapeDtypeStruct((NW, NSEG, L), jnp.float32),
    ],
    mesh=_MESH,
    compiler_params=pltpu.CompilerParams(needs_layout_passes=False),
    scratch_types=[
        pltpu.VMEM((PER_W,), jnp.int32),
        pltpu.VMEM((CH, D), jnp.float32),
        pltpu.VMEM((CH, D), jnp.float32),
        pltpu.VMEM((NSEG, D), jnp.float32),
        pltpu.VMEM((NSEG, L), jnp.float32),
        pltpu.SemaphoreType.DMA,
        pltpu.SemaphoreType.DMA,
        pltpu.SemaphoreType.DMA,
    ],
)
def _sc_segsum(x_hbm, b_hbm, sums_out, cnt_out,
               idx_v, xa, xb, acc, cnt, sem_a, sem_b, sem_i):
    cid = lax.axis_index("c")
    sid = lax.axis_index("s")
    wid = sid * NC + cid
    base = wid * PER_W

    z16 = jnp.zeros((L,), jnp.float32)
    o16 = jnp.ones((L,), jnp.float32)

    idx_cp = pltpu.async_copy(b_hbm.at[pl.ds(base, PER_W)], idx_v, sem_i)
    pltpu.async_copy(x_hbm.at[pl.ds(base, CH)], xa, sem_a)
    pltpu.async_copy(x_hbm.at[pl.ds(base + CH, CH)], xb, sem_b)

    def _zero(i, _):
        for cc in range(D // L):
            acc[i, pl.ds(cc * L, L)] = z16
        cnt[i, :] = z16
        return _

    lax.fori_loop(0, NSEG, _zero, None)
    idx_cp.wait()

    iota16 = lax.iota(jnp.int32, L)

    def _chunk(cur, idx_off):
        def grp(g, _):
            ids16 = idx_v[pl.ds(idx_off + g * L, L)]
            for u in range(L):
                r = g * L + u
                bid = ids16[jnp.full((L,), u, jnp.int32)]
                ngrp = D // L // 8

                def _ld(cc8):
                    return [cur[r, pl.ds((cc8 * 8 + k) * L, L)]
                            for k in range(8)]

                def _st(cc8, ps):
                    for k in range(8):
                        col = iota16 + (cc8 * 8 + k) * L
                        plsc.addupdate_scatter(acc, [bid, col], ps[k])

                prev = _ld(0)
                for cc8 in range(1, ngrp):
                    nxt = _ld(cc8)
                    _st(cc8 - 1, prev)
                    prev = nxt
                _st(ngrp - 1, prev)
                plsc.addupdate_scatter(cnt, [bid, iota16], o16)
            return _

        lax.fori_loop(0, CH // L, grp, None)

    def _pair(p, _):
        c0 = 2 * p
        pltpu.make_async_copy(x_hbm.at[pl.ds(0, CH)], xa, sem_a).wait()
        _chunk(xa, c0 * CH)

        @pl.when(c0 + 2 < NCH)
        def _prefetch_a():
            pltpu.async_copy(
                x_hbm.at[pl.ds(base + (c0 + 2) * CH, CH)], xa, sem_a)

        pltpu.make_async_copy(x_hbm.at[pl.ds(0, CH)], xb, sem_b).wait()
        _chunk(xb, (c0 + 1) * CH)

        @pl.when(c0 + 3 < NCH)
        def _prefetch_b():
            pltpu.async_copy(
                x_hbm.at[pl.ds(base + (c0 + 3) * CH, CH)], xb, sem_b)

        return _

    lax.fori_loop(0, NCH // 2, _pair, None)

    pltpu.sync_copy(acc, sums_out.at[wid])
    pltpu.sync_copy(cnt, cnt_out.at[wid])


def _tc_body(b_ref, x_ref, acc_out, cnt_out, accs, cnts):
    i = pl.program_id(0)

    @pl.when(i == 0)
    def _init():
        accs[...] = jnp.zeros_like(accs)
        cnts[...] = jnp.zeros_like(cnts)

    b = b_ref[0, 0, :]
    seg = lax.broadcasted_iota(jnp.int32, (NSEG, R), 0)
    pos = lax.broadcasted_iota(jnp.int32, (NSEG, R), 1)
    valid = (pos + (i + B0) * R) >= SC_ROWS
    oh = ((seg == b[None, :]) & valid).astype(jnp.float32)
    accs[...] += jnp.dot(oh, x_ref[...], preferred_element_type=jnp.float32)
    cnts[...] += jnp.sum(oh, axis=1, keepdims=True)

    @pl.when(i == pl.num_programs(0) - 1)
    def _finish():
        acc_out[...] = accs[...]
        cnt_out[...] = cnts[...]


def _comb_body(sums_sc_ref, cnt_sc_ref, acc_tc_ref, cnt_tc_ref, o_ref):
    s = jnp.sum(sums_sc_ref[...], axis=0) + acc_tc_ref[...]
    c = jnp.sum(cnt_sc_ref[...], axis=0)[:, 0:1] + cnt_tc_ref[...]
    o_ref[...] = s / jnp.maximum(c, 1.0)


def kernel(x, batch):
    b32 = batch.astype(jnp.int32)
    b3 = b32.reshape(ROWS // R, 1, R)
    acc_tc, cnt_tc = pl.pallas_call(
        _tc_body,
        grid=(ROWS // R - B0,),
        in_specs=[
            pl.BlockSpec((1, 1, R), lambda i: (i + B0, 0, 0)),
            pl.BlockSpec((R, D), lambda i: (i + B0, 0)),
        ],
        out_specs=[
            pl.BlockSpec((NSEG, D), lambda i: (0, 0)),
            pl.BlockSpec((NSEG, 1), lambda i: (0, 0)),
        ],
        out_shape=[
            jax.ShapeDtypeStruct((NSEG, D), jnp.float32),
            jax.ShapeDtypeStruct((NSEG, 1), jnp.float32),
        ],
        scratch_shapes=[
            pltpu.VMEM((NSEG, D), jnp.float32),
            pltpu.VMEM((NSEG, 1), jnp.float32),
        ],
    )(b3, x)
    sums_sc, cnt_sc = _sc_segsum(x, b32)
    return pl.pallas_call(
        _comb_body,
        grid=(1,),
        in_specs=[
            pl.BlockSpec((NW, NSEG, D), lambda i: (0, 0, 0)),
            pl.BlockSpec((NW, NSEG, L), lambda i: (0, 0, 0)),
            pl.BlockSpec((NSEG, D), lambda i: (0, 0)),
            pl.BlockSpec((NSEG, 1), lambda i: (0, 0)),
        ],
        out_specs=pl.BlockSpec((NSEG, D), lambda i: (0, 0)),
        out_shape=jax.ShapeDtypeStruct((NSEG, D), jnp.float32),
    )(sums_sc, cnt_sc, acc_tc, cnt_tc)
